# Initial kernel scaffold; baseline (speedup 1.0000x reference)
import jax
import jax.numpy as jnp
from jax.experimental import pallas as pl


def kernel(x_r1, x_r2, x_i1, x_i2, edge_index, W1, W2, b_r1, b_r2, b_i1, b_i2):
    raise NotImplementedError("write your pallas kernel here")


# speedup vs baseline: 9.4163x; 9.4163x over previous
import functools

import jax
import jax.numpy as jnp
from jax import lax
from jax.experimental import pallas as pl
from jax.experimental.pallas import tpu as pltpu
from jax.experimental.pallas import tpu_sc as plsc

N = 10000
NP = 10240
E = 320000
D = 128
NSUB = 16
NTILE = 32
B = 80
ROWS_PER_TILE = NP // NSUB
EP_MAIN = E // NSUB
EP_DEG = E // NTILE
BN = 640


def _deg_kernel_body(dst_hbm, ones_hbm, zeros_hbm, out_hbm, dbuf, ones_v, acc):
    co = lax.axis_index("c")
    s = lax.axis_index("s")
    tid = co * NSUB + s
    r0 = s * ROWS_PER_TILE
    pltpu.sync_copy(zeros_hbm.at[pl.ds(r0, ROWS_PER_TILE)],
                    acc.at[pl.ds(r0, ROWS_PER_TILE)])
    pltpu.sync_copy(ones_hbm, ones_v)
    plsc.subcore_barrier()
    base = tid * EP_DEG

    def body(i, carry):
        pltpu.sync_copy(dst_hbm.at[pl.ds(base + i * B, B)], dbuf)
        pltpu.sync_copy(ones_v, acc.at[dbuf], add=True)
        return carry

    lax.fori_loop(0, EP_DEG // B, body, 0)
    plsc.subcore_barrier()
    pltpu.sync_copy(acc.at[pl.ds(r0, ROWS_PER_TILE)],
                    out_hbm.at[co, pl.ds(r0, ROWS_PER_TILE)])


def _main_kernel_body(hp_hbm, src_hbm, dst_hbm, out_hbm,
                      sbuf, dbuf, gbuf, rows, acc):
    co = lax.axis_index("c")
    s = lax.axis_index("s")
    r0 = s * ROWS_PER_TILE
    base = s * EP_MAIN
    for k in range(2):
        chunk = co * 2 + k
        pltpu.sync_copy(hp_hbm.at[pl.ds(chunk * NP + r0, ROWS_PER_TILE)],
                        acc.at[pl.ds(r0, ROWS_PER_TILE)])
        plsc.subcore_barrier()

        def body(i, carry):
            off = base + i * B
            pltpu.sync_copy(src_hbm.at[pl.ds(off, B)], sbuf)
            pltpu.sync_copy(dst_hbm.at[pl.ds(off, B)], dbuf)
            for j in range(B // 16):
                gbuf[pl.ds(j * 16, 16)] = sbuf[pl.ds(j * 16, 16)] + chunk * NP
            pltpu.sync_copy(hp_hbm.at[gbuf], rows)
            pltpu.sync_copy(rows, acc.at[dbuf], add=True)
            return carry

        lax.fori_loop(0, EP_MAIN // B, body, 0)
        plsc.subcore_barrier()
        pltpu.sync_copy(acc.at[pl.ds(r0, ROWS_PER_TILE)],
                        out_hbm.at[chunk, pl.ds(r0, ROWS_PER_TILE)])


def _tc1_body(x_ref, w_ref, deg_ref, o_ref):
    x = x_ref[0]
    w = w_ref[0]
    dp = deg_ref[...]
    deg = dp[0, :, 0:1] + dp[1, :, 0:1] + 1.0
    dinv = lax.rsqrt(deg)
    h = lax.dot_general(x, w, (((1,), (1,)), ((), ())),
                        preferred_element_type=jnp.float32)
    o_ref[0] = h * dinv


def _tc2_body(acc_ref, deg_ref, b_ref, o_ref):
    dp = deg_ref[...]
    deg = dp[0, :, 0:1] + dp[1, :, 0:1] + 1.0
    dinv = lax.rsqrt(deg)
    o_ref[0] = acc_ref[0] * dinv + b_ref[0]


def kernel(x_r1, x_r2, x_i1, x_i2, edge_index, W1, W2, b_r1, b_r2, b_i1, b_i2):
    src = edge_index[0]
    dst = edge_index[1]
    x4 = jnp.stack([x_r1, x_r2, x_i1, x_i2])
    ws = jnp.stack([W1, W2, W1, W2])
    b4 = jnp.stack([b_r1, b_r2, b_i1, b_i2]).reshape(4, 1, D)
    ones16 = jnp.ones((B, 16), jnp.float32)
    zeros16 = jnp.zeros((NP, 16), jnp.float32)

    mesh = plsc.VectorSubcoreMesh(core_axis_name="c", subcore_axis_name="s")

    deg_kernel = functools.partial(
        pl.kernel,
        mesh=mesh,
        out_type=jax.ShapeDtypeStruct((2, NP, 16), jnp.float32),
        scratch_types=[
            pltpu.VMEM((B,), jnp.int32),
            pltpu.VMEM((B, 16), jnp.float32),
            pltpu.VMEM_SHARED((NP, 16), jnp.float32),
        ],
    )(_deg_kernel_body)
    degp = deg_kernel(dst, ones16, zeros16)

    nb = NP // BN
    hp = pl.pallas_call(
        _tc1_body,
        grid=(4, nb),
        in_specs=[
            pl.BlockSpec((1, BN, D), lambda c, i: (c, i, 0)),
            pl.BlockSpec((1, D, D), lambda c, i: (c, 0, 0)),
            pl.BlockSpec((2, BN, 16), lambda c, i: (0, i, 0)),
        ],
        out_specs=pl.BlockSpec((1, BN, D), lambda c, i: (c, i, 0)),
        out_shape=jax.ShapeDtypeStruct((4, NP, D), jnp.float32),
    )(x4, ws, degp)

    main_kernel = functools.partial(
        pl.kernel,
        mesh=mesh,
        out_type=jax.ShapeDtypeStruct((4, NP, D), jnp.float32),
        scratch_types=[
            pltpu.VMEM((B,), jnp.int32),
            pltpu.VMEM((B,), jnp.int32),
            pltpu.VMEM((B,), jnp.int32),
            pltpu.VMEM((B, D), jnp.float32),
            pltpu.VMEM_SHARED((NP, D), jnp.float32),
        ],
    )(_main_kernel_body)
    accp = main_kernel(hp.reshape(4 * NP, D), src, dst)

    out = pl.pallas_call(
        _tc2_body,
        grid=(4, nb),
        in_specs=[
            pl.BlockSpec((1, BN, D), lambda c, i: (c, i, 0)),
            pl.BlockSpec((2, BN, 16), lambda c, i: (0, i, 0)),
            pl.BlockSpec((1, 1, D), lambda c, i: (c, 0, 0)),
        ],
        out_specs=pl.BlockSpec((1, BN, D), lambda c, i: (c, i, 0)),
        out_shape=jax.ShapeDtypeStruct((4, NP, D), jnp.float32),
    )(accp, degp, b4)

    return (out[0, :N], out[1, :N], out[2, :N], out[3, :N])


# speedup vs baseline: 17.8977x; 1.9007x over previous
import functools

import jax
import jax.numpy as jnp
from jax import lax
from jax.experimental import pallas as pl
from jax.experimental.pallas import tpu as pltpu
from jax.experimental.pallas import tpu_sc as plsc

N = 10000
NP = 10240
E = 320000
D = 128
NSUB = 16
NTILE = 32
B = 80
ROWS_PER_TILE = NP // NSUB
EP_MAIN = E // NSUB
EP_DEG = E // NTILE
BN = 640


def _deg_kernel_body(dst_hbm, ones_hbm, zeros_hbm, out_hbm, dbuf, ones_v, acc):
    co = lax.axis_index("c")
    s = lax.axis_index("s")
    tid = co * NSUB + s
    r0 = s * ROWS_PER_TILE
    pltpu.sync_copy(zeros_hbm.at[pl.ds(r0, ROWS_PER_TILE)],
                    acc.at[pl.ds(r0, ROWS_PER_TILE)])
    pltpu.sync_copy(ones_hbm, ones_v)
    pltpu.sync_copy(dst_hbm.at[tid], dbuf)
    plsc.subcore_barrier()

    def body(i, carry):
        pltpu.sync_copy(ones_v, acc.at[dbuf.at[i]], add=True)
        return carry

    lax.fori_loop(0, EP_DEG // B, body, 0)
    plsc.subcore_barrier()
    pltpu.sync_copy(acc.at[pl.ds(r0, ROWS_PER_TILE)],
                    out_hbm.at[co, pl.ds(r0, ROWS_PER_TILE)])


def _main_kernel_body(hp_hbm, sdx_hbm, out_hbm,
                      sd_a, sd_b, rows_a, rows_b, acc,
                      sem_ga, sem_gb, sem_ia, sem_ib):
    co = lax.axis_index("c")
    s = lax.axis_index("s")
    r0 = s * ROWS_PER_TILE
    nb_t = EP_MAIN // B
    for k in range(2):
        chunk = co * 2 + k
        pltpu.sync_copy(hp_hbm.at[pl.ds(chunk * NP + r0, ROWS_PER_TILE)],
                        acc.at[pl.ds(r0, ROWS_PER_TILE)])
        plsc.subcore_barrier()

        pltpu.sync_copy(sdx_hbm.at[chunk, s, 0], sd_a)
        pltpu.make_async_copy(hp_hbm.at[sd_a.at[0]], rows_a, sem_ga).start()
        pltpu.make_async_copy(
            sdx_hbm.at[chunk, s, 1], sd_b, sem_ib).start()

        def body(i, carry):
            last = nb_t // 2 - 1
            pltpu.make_async_copy(
                sdx_hbm.at[chunk, s, 2 * i + 1], sd_b, sem_ib).wait()
            pltpu.make_async_copy(
                hp_hbm.at[sd_b.at[0]], rows_b, sem_gb).start()
            pltpu.make_async_copy(
                hp_hbm.at[sd_a.at[0]], rows_a, sem_ga).wait()
            pltpu.sync_copy(rows_a, acc.at[sd_a.at[1]], add=True)

            @pl.when(i < last)
            def _():
                pltpu.make_async_copy(
                    sdx_hbm.at[chunk, s, 2 * i + 2], sd_a, sem_ia).start()

            pltpu.make_async_copy(
                hp_hbm.at[sd_b.at[0]], rows_b, sem_gb).wait()
            pltpu.sync_copy(rows_b, acc.at[sd_b.at[1]], add=True)

            @pl.when(i < last)
            def _():
                pltpu.make_async_copy(
                    sdx_hbm.at[chunk, s, 2 * i + 2], sd_a, sem_ia).wait()
                pltpu.make_async_copy(
                    hp_hbm.at[sd_a.at[0]], rows_a, sem_ga).start()
                pltpu.make_async_copy(
                    sdx_hbm.at[chunk, s, 2 * i + 3], sd_b, sem_ib).start()

            return carry

        lax.fori_loop(0, nb_t // 2, body, 0)
        plsc.subcore_barrier()
        pltpu.sync_copy(acc.at[pl.ds(r0, ROWS_PER_TILE)],
                        out_hbm.at[chunk, pl.ds(r0, ROWS_PER_TILE)])


def _tc1_body(x_ref, w_ref, deg_ref, o_ref):
    x = x_ref[0]
    w = w_ref[0]
    dp = deg_ref[...]
    deg = dp[0, :, 0:1] + dp[1, :, 0:1] + 1.0
    dinv = lax.rsqrt(deg)
    h = lax.dot_general(x, w, (((1,), (1,)), ((), ())),
                        preferred_element_type=jnp.float32)
    o_ref[0] = h * dinv


def _tc2_body(acc_ref, deg_ref, b_ref, o_ref):
    dp = deg_ref[...]
    deg = dp[0, :, 0:1] + dp[1, :, 0:1] + 1.0
    dinv = lax.rsqrt(deg)
    o_ref[0] = acc_ref[0] * dinv + b_ref[0]


def kernel(x_r1, x_r2, x_i1, x_i2, edge_index, W1, W2, b_r1, b_r2, b_i1, b_i2):
    src = edge_index[0]
    dst = edge_index[1]
    x4 = jnp.stack([x_r1, x_r2, x_i1, x_i2])
    ws = jnp.stack([W1, W2, W1, W2])
    b4 = jnp.stack([b_r1, b_r2, b_i1, b_i2]).reshape(4, 1, D)
    ones16 = jnp.ones((B, 16), jnp.float32)
    zeros16 = jnp.zeros((NP, 16), jnp.float32)

    mesh = plsc.VectorSubcoreMesh(core_axis_name="c", subcore_axis_name="s")

    deg_kernel = functools.partial(
        pl.kernel,
        mesh=mesh,
        out_type=jax.ShapeDtypeStruct((2, NP, 16), jnp.float32),
        scratch_types=[
            pltpu.VMEM((EP_DEG // B, B), jnp.int32),
            pltpu.VMEM((B, 16), jnp.float32),
            pltpu.VMEM_SHARED((NP, 16), jnp.float32),
        ],
    )(_deg_kernel_body)
    dstd = dst.reshape(NTILE, EP_DEG // B, B)
    degp = deg_kernel(dstd, ones16, zeros16)

    nb = NP // BN
    hp = pl.pallas_call(
        _tc1_body,
        grid=(4, nb),
        in_specs=[
            pl.BlockSpec((1, BN, D), lambda c, i: (c, i, 0)),
            pl.BlockSpec((1, D, D), lambda c, i: (c, 0, 0)),
            pl.BlockSpec((2, BN, 16), lambda c, i: (0, i, 0)),
        ],
        out_specs=pl.BlockSpec((1, BN, D), lambda c, i: (c, i, 0)),
        out_shape=jax.ShapeDtypeStruct((4, NP, D), jnp.float32),
    )(x4, ws, degp)

    main_kernel = functools.partial(
        pl.kernel,
        mesh=mesh,
        out_type=jax.ShapeDtypeStruct((4, NP, D), jnp.float32),
        scratch_types=[
            pltpu.VMEM((2, B), jnp.int32),
            pltpu.VMEM((2, B), jnp.int32),
            pltpu.VMEM((B, D), jnp.float32),
            pltpu.VMEM((B, D), jnp.float32),
            pltpu.VMEM_SHARED((NP, D), jnp.float32),
            pltpu.SemaphoreType.DMA,
            pltpu.SemaphoreType.DMA,
            pltpu.SemaphoreType.DMA,
            pltpu.SemaphoreType.DMA,
        ],
    )(_main_kernel_body)
    nb_t = EP_MAIN // B
    srcp = (src[None, :] + (NP * jnp.arange(4, dtype=jnp.int32))[:, None])
    srcp4 = srcp.reshape(4, NSUB, nb_t, 1, B)
    dst4 = jnp.broadcast_to(dst.reshape(1, NSUB, nb_t, 1, B),
                            (4, NSUB, nb_t, 1, B))
    sdx = jnp.concatenate([srcp4, dst4], axis=3)
    accp = main_kernel(hp.reshape(4 * NP, D), sdx)

    out = pl.pallas_call(
        _tc2_body,
        grid=(4, nb),
        in_specs=[
            pl.BlockSpec((1, BN, D), lambda c, i: (c, i, 0)),
            pl.BlockSpec((2, BN, 16), lambda c, i: (0, i, 0)),
            pl.BlockSpec((1, 1, D), lambda c, i: (c, 0, 0)),
        ],
        out_specs=pl.BlockSpec((1, BN, D), lambda c, i: (c, i, 0)),
        out_shape=jax.ShapeDtypeStruct((4, NP, D), jnp.float32),
    )(accp, degp, b4)

    return (out[0, :N], out[1, :N], out[2, :N], out[3, :N])
